# baseline (speedup 1.0000x reference)
import dataclasses
import functools

import jax
import jax.numpy as jnp
from jax import lax
from jax.experimental import pallas as pl
from jax.experimental.pallas import tpu as pltpu
from jax.experimental.pallas import tpu_sc as plsc

NC = 2
NS = 16
NW = NC * NS
LANES = 16
BLK = 32
RING = 4


def _sc_compiler_params():
    cp = pltpu.CompilerParams(use_tc_tiling_on_sc=False)
    if "needs_layout_passes" in pltpu.CompilerParams.__dataclass_fields__:
        cp = dataclasses.replace(cp, needs_layout_passes=False)
    return cp


def _tree(vals):
    while len(vals) > 1:
        vals = [a + b for a, b in zip(vals[::2], vals[1::2])]
    return vals[0]


def _sc_scores(u_weight, v_weight, pos_u, pos_v, neg_v, B, D, S):
    bpw = B // NW
    nblk = bpw // BLK
    neg = S - 1
    nch = D // LANES
    pu = pos_u.reshape(NW, nblk, BLK)
    pv = pos_v.reshape(NW, nblk, BLK)
    nv = neg_v.reshape(NW, nblk, neg * BLK)

    mesh = plsc.VectorSubcoreMesh(core_axis_name="c", subcore_axis_name="s")

    @functools.partial(
        pl.kernel,
        mesh=mesh,
        compiler_params=_sc_compiler_params(),
        out_type=jax.ShapeDtypeStruct((NW * S, bpw), jnp.float32),
        scratch_types=(
            [pltpu.VMEM((nblk, BLK), jnp.int32),
             pltpu.VMEM((nblk, S * BLK), jnp.int32)]
            + [pltpu.VMEM((BLK, D), jnp.float32) for _ in range(RING)]
            + [pltpu.VMEM((S * BLK, D), jnp.float32) for _ in range(RING)]
            + [pltpu.VMEM((S, bpw), jnp.float32),
               pltpu.VMEM((S, BLK, LANES + 1), jnp.float32)]
            + [pltpu.SemaphoreType.DMA for _ in range(RING)]
        ),
    )
    def k(uw_hbm, vw_hbm, pu_hbm, pv_hbm, nv_hbm, scores_hbm,
          idxu_v, idxv_v, *rest):
        ubufs = rest[:RING]
        vbufs = rest[RING:2 * RING]
        scores_v, part_v = rest[2 * RING], rest[2 * RING + 1]
        sems = rest[2 * RING + 2:]

        wid = lax.axis_index("s") * NC + lax.axis_index("c")
        pltpu.sync_copy(pu_hbm.at[wid], idxu_v)
        pltpu.sync_copy(pv_hbm.at[wid], idxv_v.at[:, pl.ds(0, BLK)])
        pltpu.sync_copy(nv_hbm.at[wid], idxv_v.at[:, pl.ds(BLK, neg * BLK)])

        def start(b, s):
            pltpu.async_copy(uw_hbm.at[idxu_v.at[b]], ubufs[s], sems[s])
            pltpu.async_copy(vw_hbm.at[idxv_v.at[b]], vbufs[s], sems[s])

        def wait(b, s):
            pltpu.make_async_copy(
                uw_hbm.at[idxu_v.at[b]], ubufs[s], sems[s]).wait()
            pltpu.make_async_copy(
                vw_hbm.at[idxv_v.at[b]], vbufs[s], sems[s]).wait()

        def compute(bb, s):
            ub, vb = ubufs[s], vbufs[s]

            @pl.loop(0, BLK)
            def _(e):
                u = [ub[e, pl.ds(c * LANES, LANES)] for c in range(nch)]
                for j in range(S):
                    r = e if j == 0 else BLK + e * neg + (j - 1)
                    p = _tree([vb[r, pl.ds(c * LANES, LANES)] * u[c]
                               for c in range(nch)])
                    part_v[j, e, pl.ds(0, LANES)] = -p if j == 0 else p

            for g in range(BLK // LANES):
                rows = jnp.arange(LANES, dtype=jnp.int32) + (g * LANES)
                for j in range(S):
                    jcol = jnp.full((LANES,), j, jnp.int32)
                    cols = [
                        plsc.load_gather(
                            part_v,
                            [jcol, rows, jnp.full((LANES,), l, jnp.int32)])
                        for l in range(LANES)
                    ]
                    scores_v[j, pl.ds(bb * BLK + g * LANES, LANES)] = (
                        _tree(cols))

        for r in range(RING):
            start(r, r)

        @pl.loop(0, nblk, step=RING)
        def _(b):
            for s in range(RING):
                bb = b + s
                wait(bb, s)
                compute(bb, s)

                @pl.when(bb + RING < nblk)
                def _():
                    start(bb + RING, s)

        pltpu.sync_copy(scores_v, scores_hbm.at[pl.ds(wid * S, S)])

    return k(u_weight, v_weight, pu, pv, nv)


def _tc_loss(scores):
    def body(s_ref, out_ref):
        s = jnp.clip(s_ref[...], -10.0, 10.0)
        total = jnp.sum(jnp.log1p(jnp.exp(s)))
        out_ref[...] = jnp.full((1, 1), 0.0, jnp.float32) + total

    out = pl.pallas_call(
        body,
        out_shape=jax.ShapeDtypeStruct((1, 1), jnp.float32),
    )(scores)
    return out[0, 0]


def kernel(pos_u, pos_v, neg_v, u_weight, v_weight):
    B = pos_u.shape[0]
    D = u_weight.shape[1]
    S = neg_v.shape[1] + 1
    scores = _sc_scores(u_weight, v_weight, pos_u, pos_v, neg_v, B, D, S)
    return _tc_loss(scores) / B


# speedup vs baseline: 1.0660x; 1.0660x over previous
import dataclasses
import functools

import jax
import jax.numpy as jnp
from jax import lax
from jax.experimental import pallas as pl
from jax.experimental.pallas import tpu as pltpu
from jax.experimental.pallas import tpu_sc as plsc

NC = 2
NS = 16
NW = NC * NS
LANES = 16
BLK = 32


def _sc_compiler_params():
    cp = pltpu.CompilerParams(use_tc_tiling_on_sc=False)
    if "needs_layout_passes" in pltpu.CompilerParams.__dataclass_fields__:
        cp = dataclasses.replace(cp, needs_layout_passes=False)
    return cp


def _tree(vals):
    while len(vals) > 1:
        vals = [a + b for a, b in zip(vals[::2], vals[1::2])]
    return vals[0]


def _sc_phase(u_weight, v_weight, ufi, uti, vfi, vti, B, D, S, EF):
    bpw = B // NW
    ET = bpw - EF
    npairs = EF // BLK
    assert ET == EF
    neg = S - 1
    nch = D // LANES
    BT = NW * ET

    mesh = plsc.VectorSubcoreMesh(core_axis_name="c", subcore_axis_name="s")

    @functools.partial(
        pl.kernel,
        mesh=mesh,
        compiler_params=_sc_compiler_params(),
        out_type=[
            jax.ShapeDtypeStruct((NW * S, EF), jnp.float32),
            jax.ShapeDtypeStruct((BT, D), jnp.float32),
            jax.ShapeDtypeStruct((S, BT, D), jnp.float32),
        ],
        scratch_types=(
            [pltpu.VMEM((npairs, BLK), jnp.int32),
             pltpu.VMEM((npairs, BLK), jnp.int32),
             pltpu.VMEM((npairs, S * BLK), jnp.int32),
             pltpu.VMEM((npairs, S * BLK), jnp.int32)]
            + [pltpu.VMEM((BLK, D), jnp.float32) for _ in range(4)]
            + [pltpu.VMEM((S * BLK, D), jnp.float32) for _ in range(4)]
            + [pltpu.VMEM((S, EF), jnp.float32),
               pltpu.VMEM((S, BLK, LANES + 1), jnp.float32)]
            + [pltpu.SemaphoreType.DMA for _ in range(6)]
        ),
    )
    def k(uw_hbm, vw_hbm, ufi_hbm, uti_hbm, vfi_hbm, vti_hbm,
          scores_hbm, ut_hbm, vt_hbm,
          ufi_v, uti_v, vfi_v, vti_v, *rest):
        ubf = rest[0:2]
        ubt = rest[2:4]
        vbf = rest[4:6]
        vbt = rest[6:8]
        scores_v, part_v = rest[8], rest[9]
        semf = rest[10:12]
        semt = rest[12:14]
        semw = rest[14:16]

        wid = lax.axis_index("s") * NC + lax.axis_index("c")
        pltpu.sync_copy(ufi_hbm.at[wid], ufi_v)
        pltpu.sync_copy(uti_hbm.at[wid], uti_v)
        pltpu.sync_copy(vfi_hbm.at[wid], vfi_v)
        pltpu.sync_copy(vti_hbm.at[wid], vti_v)

        def startf(p, s):
            pltpu.async_copy(uw_hbm.at[ufi_v.at[p]], ubf[s], semf[s])
            pltpu.async_copy(vw_hbm.at[vfi_v.at[p]], vbf[s], semf[s])

        def waitf(p, s):
            pltpu.make_async_copy(
                uw_hbm.at[ufi_v.at[p]], ubf[s], semf[s]).wait()
            pltpu.make_async_copy(
                vw_hbm.at[vfi_v.at[p]], vbf[s], semf[s]).wait()

        def startt(p, s):
            pltpu.async_copy(uw_hbm.at[uti_v.at[p]], ubt[s], semt[s])
            pltpu.async_copy(vw_hbm.at[vti_v.at[p]], vbt[s], semt[s])

        def waitt(p, s):
            pltpu.make_async_copy(
                uw_hbm.at[uti_v.at[p]], ubt[s], semt[s]).wait()
            pltpu.make_async_copy(
                vw_hbm.at[vti_v.at[p]], vbt[s], semt[s]).wait()

        def wb_copies(p, s):
            off = wid * ET + p * BLK
            yield ubt[s], ut_hbm.at[pl.ds(off, BLK)]
            for j in range(S):
                yield (vbt[s].at[pl.ds(j * BLK, BLK)],
                       vt_hbm.at[j, pl.ds(off, BLK)])

        def start_wb(p, s):
            for src, dst in wb_copies(p, s):
                pltpu.async_copy(src, dst, semw[s])

        def wait_wb(p, s):
            for src, dst in wb_copies(p, s):
                pltpu.make_async_copy(src, dst, semw[s]).wait()

        def compute(p, s):
            ub, vb = ubf[s], vbf[s]

            @pl.loop(0, BLK)
            def _(e):
                u = [ub[e, pl.ds(c * LANES, LANES)] for c in range(nch)]
                for j in range(S):
                    r = e if j == 0 else BLK + e * neg + (j - 1)
                    prod = _tree([vb[r, pl.ds(c * LANES, LANES)] * u[c]
                                  for c in range(nch)])
                    part_v[j, e, pl.ds(0, LANES)] = (
                        -prod if j == 0 else prod)

            for g in range(BLK // LANES):
                rows = jnp.arange(LANES, dtype=jnp.int32) + (g * LANES)
                for j in range(S):
                    jcol = jnp.full((LANES,), j, jnp.int32)
                    cols = [
                        plsc.load_gather(
                            part_v,
                            [jcol, rows, jnp.full((LANES,), l, jnp.int32)])
                        for l in range(LANES)
                    ]
                    scores_v[j, pl.ds(p * BLK + g * LANES, LANES)] = (
                        _tree(cols))

        startf(0, 0)
        startt(0, 0)

        @pl.loop(0, npairs, step=2)
        def _(p0):
            for par in range(2):
                p = p0 + par
                nxt = (par + 1) % 2

                @pl.when(p + 1 < npairs)
                def _():
                    startf(p + 1, nxt)

                waitf(p, par)
                compute(p, par)

                waitt(p, par)
                start_wb(p, par)

                @pl.when(p + 1 < npairs)
                def _():
                    @pl.when(p >= 1)
                    def _():
                        wait_wb(p - 1, nxt)

                    startt(p + 1, nxt)

        wait_wb(npairs - 1, (npairs - 1) % 2)
        pltpu.sync_copy(scores_v, scores_hbm.at[pl.ds(wid * S, S)])

    return k(u_weight, v_weight, ufi, uti, vfi, vti)


def _tc_loss(emb_u, emb_v6, scores, BT, D, nb):
    def body(u_ref, v6_ref, sc_ref, out_ref):
        i = pl.program_id(0)

        @pl.when(i == 0)
        def _():
            sc = jnp.clip(sc_ref[...], -10.0, 10.0)
            out_ref[...] = (jnp.zeros((1, 1), jnp.float32)
                            + jnp.sum(jnp.log1p(jnp.exp(sc))))

        u = u_ref[...]
        s = jnp.sum(u * v6_ref[0], axis=1)
        s = jnp.clip(s, -10.0, 10.0)
        loss = jnp.log1p(jnp.exp(-s))
        for j in range(1, 6):
            t = jnp.sum(u * v6_ref[j], axis=1)
            t = jnp.clip(t, -10.0, 10.0)
            loss = loss + jnp.log1p(jnp.exp(t))
        out_ref[...] = out_ref[...] + jnp.sum(loss)

    out = pl.pallas_call(
        body,
        grid=(BT // nb,),
        in_specs=[
            pl.BlockSpec((nb, D), lambda i: (i, 0)),
            pl.BlockSpec((6, nb, D), lambda i: (0, i, 0)),
            pl.BlockSpec(scores.shape, lambda i: (0, 0)),
        ],
        out_specs=pl.BlockSpec((1, 1), lambda i: (0, 0)),
        out_shape=jax.ShapeDtypeStruct((1, 1), jnp.float32),
    )(emb_u, emb_v6, scores)
    return out[0, 0]


def kernel(pos_u, pos_v, neg_v, u_weight, v_weight):
    B = pos_u.shape[0]
    D = u_weight.shape[1]
    S = neg_v.shape[1] + 1
    bpw = B // NW
    EF = bpw // 2
    npairs = EF // BLK

    pu = pos_u.reshape(NW, bpw)
    ufi = pu[:, :EF].reshape(NW, npairs, BLK)
    uti = pu[:, EF:].reshape(NW, npairs, BLK)

    pv = pos_v.reshape(NW, bpw)
    nv = neg_v.reshape(NW, bpw, S - 1)
    vfi = jnp.concatenate(
        [pv[:, :EF].reshape(NW, npairs, BLK),
         nv[:, :EF].reshape(NW, npairs, (S - 1) * BLK)], axis=-1)

    c6 = jnp.concatenate([pos_v[None, :], neg_v.T], axis=0)
    vti = (c6.reshape(S, NW, bpw)[:, :, EF:]
           .reshape(S, NW, npairs, BLK)
           .transpose(1, 2, 0, 3)
           .reshape(NW, npairs, S * BLK))

    scores, ut, vt = _sc_phase(
        u_weight, v_weight, ufi, uti, vfi, vti, B, D, S, EF)
    total = _tc_loss(ut, vt, scores, NW * (bpw - EF), D, nb=2048)
    return total / B
